# baseline (speedup 1.0000x reference)
import functools

import jax
import jax.numpy as jnp
from jax import lax
from jax.experimental import pallas as pl
from jax.experimental.pallas import tpu as pltpu
from jax.experimental.pallas import tpu_sc as plsc

_B, _L, _D = 1024, 200, 128
_NTOK = _B * _L
_NW = 32
_TOK_PER_W = _NTOK // _NW
_C = 256
_NSUB = _C // 128
_NCHUNK = _TOK_PER_W // _C
_BB = 8


def _sc_gather(ids_flat, table):
    mesh = plsc.VectorSubcoreMesh(core_axis_name="c", subcore_axis_name="s")

    @functools.partial(
        pl.kernel,
        out_type=jax.ShapeDtypeStruct((_NTOK, _D), jnp.float32),
        mesh=mesh,
        scratch_types=[
            pltpu.VMEM((_C,), jnp.int32),
            pltpu.VMEM((_C,), jnp.int32),
            pltpu.VMEM((_C, _D), jnp.float32),
            pltpu.VMEM((_C, _D), jnp.float32),
            pltpu.SemaphoreType.DMA,
            pltpu.SemaphoreType.DMA,
            pltpu.SemaphoreType.DMA,
            pltpu.SemaphoreType.DMA,
        ],
    )
    def gather_kernel(ids_hbm, table_hbm, out_hbm, idx0, idx1, rows0, rows1,
                      sg0, sg1, sw0, sw1):
        wid = lax.axis_index("s") * 2 + lax.axis_index("c")
        base = wid * _TOK_PER_W
        idx = (idx0, idx1)
        rows = (rows0, rows1)
        sg = (sg0, sg1)
        sw = (sw0, sw1)

        def off(g):
            return pl.multiple_of(base + g * _C, 8)

        def fire_gather(g):
            b = g % 2
            return [
                pltpu.async_copy(
                    table_hbm.at[idx[b].at[pl.ds(j * 128, 128)]],
                    rows[b].at[pl.ds(j * 128, 128)],
                    sg[b],
                )
                for j in range(_NSUB)
            ]

        def fire_wb(g):
            b = g % 2
            return pltpu.async_copy(rows[b], out_hbm.at[pl.ds(off(g), _C)], sw[b])

        pltpu.sync_copy(ids_hbm.at[pl.ds(off(0), _C)], idx0)
        gath = {0: fire_gather(0)}
        pltpu.sync_copy(ids_hbm.at[pl.ds(off(1), _C)], idx1)
        wb = {}
        for g in range(_NCHUNK):
            if g + 1 < _NCHUNK:
                if g - 1 in wb:
                    wb.pop(g - 1).wait()
                gath[g + 1] = fire_gather(g + 1)
            for cp in gath.pop(g):
                cp.wait()
            if g + 2 < _NCHUNK:
                pltpu.sync_copy(ids_hbm.at[pl.ds(off(g + 2), _C)], idx[g % 2])
            wb[g] = fire_wb(g)
        for g in sorted(wb):
            wb.pop(g).wait()

    return gather_kernel(ids_flat, table)


def _ln_body(w_ref, tid_ref, pos_ref, type_ref, gamma_ref, beta_ref, out_ref):
    w = w_ref[...]
    tid = tid_ref[:, 0, :]
    pos = pos_ref[...]
    t0 = type_ref[0:1, :]
    t1 = type_ref[1:2, :]
    t = jnp.where(tid[:, :, None] == 0, t0[None], t1[None])
    e = w + pos[None] + t
    mean = jnp.mean(e, axis=-1, keepdims=True)
    c = e - mean
    var = jnp.mean(c * c, axis=-1, keepdims=True)
    o = c * lax.rsqrt(var + 1e-12)
    out_ref[...] = o * gamma_ref[...][None] + beta_ref[...][None]


def _tc_ln(w, tid3, pos, typ, gamma, beta):
    return pl.pallas_call(
        _ln_body,
        out_shape=jax.ShapeDtypeStruct((_B, _L, _D), jnp.float32),
        grid=(_B // _BB,),
        in_specs=[
            pl.BlockSpec((_BB, _L, _D), lambda i: (i, 0, 0)),
            pl.BlockSpec((_BB, 1, _L), lambda i: (i, 0, 0)),
            pl.BlockSpec((_L, _D), lambda i: (0, 0)),
            pl.BlockSpec((8, _D), lambda i: (0, 0)),
            pl.BlockSpec((1, _D), lambda i: (0, 0)),
            pl.BlockSpec((1, _D), lambda i: (0, 0)),
        ],
        out_specs=pl.BlockSpec((_BB, _L, _D), lambda i: (i, 0, 0)),
    )(w, tid3, pos, typ, gamma, beta)


def kernel(input_ids, token_type_ids, word_emb, pos_emb, type_emb, ln_gamma, ln_beta):
    ids_flat = input_ids.reshape(_NTOK).astype(jnp.int32)
    w = _sc_gather(ids_flat, word_emb).reshape(_B, _L, _D)
    tid3 = token_type_ids.reshape(_B, 1, _L).astype(jnp.int32)
    typ8 = jnp.zeros((8, _D), jnp.float32).at[:2].set(type_emb)
    return _tc_ln(
        w,
        tid3,
        pos_emb[:_L],
        typ8,
        ln_gamma.reshape(1, _D),
        ln_beta.reshape(1, _D),
    )


# speedup vs baseline: 1.1651x; 1.1651x over previous
import functools

import jax
import jax.numpy as jnp
from jax import lax
from jax.experimental import pallas as pl
from jax.experimental.pallas import tpu as pltpu
from jax.experimental.pallas import tpu_sc as plsc

_B, _L, _D = 1024, 200, 128
_NTOK = _B * _L
_NW = 32
_TOK_PER_W = _NTOK // _NW
_C = 256
_NSUB = _C // 128
_NCHUNK = _TOK_PER_W // _C
_BB = 16


def _sc_gather(ids_flat, table):
    mesh = plsc.VectorSubcoreMesh(core_axis_name="c", subcore_axis_name="s")

    @functools.partial(
        pl.kernel,
        out_type=jax.ShapeDtypeStruct((_NTOK, _D), jnp.float32),
        mesh=mesh,
        scratch_types=[
            pltpu.VMEM((_C,), jnp.int32),
            pltpu.VMEM((_C,), jnp.int32),
            pltpu.VMEM((_C, _D), jnp.float32),
            pltpu.VMEM((_C, _D), jnp.float32),
            pltpu.SemaphoreType.DMA,
            pltpu.SemaphoreType.DMA,
            pltpu.SemaphoreType.DMA,
            pltpu.SemaphoreType.DMA,
        ],
    )
    def gather_kernel(ids_hbm, table_hbm, out_hbm, idx0, idx1, rows0, rows1,
                      sg0, sg1, sw0, sw1):
        wid = lax.axis_index("s") * 2 + lax.axis_index("c")
        base = wid * _TOK_PER_W
        idx = (idx0, idx1)
        rows = (rows0, rows1)
        sg = (sg0, sg1)
        sw = (sw0, sw1)

        def off(g):
            return pl.multiple_of(base + g * _C, 8)

        def fire_gather(g):
            b = g % 2
            return [
                pltpu.async_copy(
                    table_hbm.at[idx[b].at[pl.ds(j * 128, 128)]],
                    rows[b].at[pl.ds(j * 128, 128)],
                    sg[b],
                )
                for j in range(_NSUB)
            ]

        def fire_wb(g):
            b = g % 2
            return pltpu.async_copy(rows[b], out_hbm.at[pl.ds(off(g), _C)], sw[b])

        pltpu.sync_copy(ids_hbm.at[pl.ds(off(0), _C)], idx0)
        gath = {0: fire_gather(0)}
        pltpu.sync_copy(ids_hbm.at[pl.ds(off(1), _C)], idx1)
        wb = {}
        for g in range(_NCHUNK):
            if g + 1 < _NCHUNK:
                if g - 1 in wb:
                    wb.pop(g - 1).wait()
                gath[g + 1] = fire_gather(g + 1)
            for cp in gath.pop(g):
                cp.wait()
            if g + 2 < _NCHUNK:
                pltpu.sync_copy(ids_hbm.at[pl.ds(off(g + 2), _C)], idx[g % 2])
            wb[g] = fire_wb(g)
        for g in sorted(wb):
            wb.pop(g).wait()

    return gather_kernel(ids_flat, table)


def _ln_body(w_ref, tid_ref, pos_ref, type_ref, gamma_ref, beta_ref, out_ref):
    w = w_ref[...]
    tid = tid_ref[:, 0, :]
    pos = pos_ref[...]
    t0 = type_ref[0:1, :]
    t1 = type_ref[1:2, :]
    t = jnp.where(tid[:, :, None] == 0, t0[None], t1[None])
    e = w + pos[None] + t
    mean = jnp.mean(e, axis=-1, keepdims=True)
    c = e - mean
    var = jnp.mean(c * c, axis=-1, keepdims=True)
    o = c * lax.rsqrt(var + 1e-12)
    out_ref[...] = o * gamma_ref[...][None] + beta_ref[...][None]


def _tc_ln(w, tid3, pos, typ, gamma, beta):
    return pl.pallas_call(
        _ln_body,
        out_shape=jax.ShapeDtypeStruct((_B, _L, _D), jnp.float32),
        grid=(_B // _BB,),
        in_specs=[
            pl.BlockSpec((_BB, _L, _D), lambda i: (i, 0, 0)),
            pl.BlockSpec((_BB, 1, _L), lambda i: (i, 0, 0)),
            pl.BlockSpec((_L, _D), lambda i: (0, 0)),
            pl.BlockSpec((8, _D), lambda i: (0, 0)),
            pl.BlockSpec((1, _D), lambda i: (0, 0)),
            pl.BlockSpec((1, _D), lambda i: (0, 0)),
        ],
        out_specs=pl.BlockSpec((_BB, _L, _D), lambda i: (i, 0, 0)),
    )(w, tid3, pos, typ, gamma, beta)


def kernel(input_ids, token_type_ids, word_emb, pos_emb, type_emb, ln_gamma, ln_beta):
    ids_flat = input_ids.reshape(_NTOK).astype(jnp.int32)
    w = _sc_gather(ids_flat, word_emb).reshape(_B, _L, _D)
    tid3 = token_type_ids.reshape(_B, 1, _L).astype(jnp.int32)
    typ8 = jnp.zeros((8, _D), jnp.float32).at[:2].set(type_emb)
    return _tc_ln(
        w,
        tid3,
        pos_emb[:_L],
        typ8,
        ln_gamma.reshape(1, _D),
        ln_beta.reshape(1, _D),
    )


# speedup vs baseline: 1.2628x; 1.0839x over previous
import functools

import jax
import jax.numpy as jnp
from jax import lax
from jax.experimental import pallas as pl
from jax.experimental.pallas import tpu as pltpu
from jax.experimental.pallas import tpu_sc as plsc

_B, _L, _D = 1024, 200, 128
_NTOK = _B * _L
_NW = 32
_TOK_PER_W = _NTOK // _NW
_C = 256
_NSUB = _C // 128
_NCHUNK = _TOK_PER_W // _C
_BB = 32


def _sc_gather(ids_flat, table):
    mesh = plsc.VectorSubcoreMesh(core_axis_name="c", subcore_axis_name="s")

    @functools.partial(
        pl.kernel,
        out_type=jax.ShapeDtypeStruct((_NTOK, _D), jnp.float32),
        mesh=mesh,
        scratch_types=[
            pltpu.VMEM((_C,), jnp.int32),
            pltpu.VMEM((_C,), jnp.int32),
            pltpu.VMEM((_C, _D), jnp.float32),
            pltpu.VMEM((_C, _D), jnp.float32),
            pltpu.SemaphoreType.DMA,
            pltpu.SemaphoreType.DMA,
            pltpu.SemaphoreType.DMA,
            pltpu.SemaphoreType.DMA,
        ],
    )
    def gather_kernel(ids_hbm, table_hbm, out_hbm, idx0, idx1, rows0, rows1,
                      sg0, sg1, sw0, sw1):
        wid = lax.axis_index("s") * 2 + lax.axis_index("c")
        base = wid * _TOK_PER_W
        idx = (idx0, idx1)
        rows = (rows0, rows1)
        sg = (sg0, sg1)
        sw = (sw0, sw1)

        def off(g):
            return pl.multiple_of(base + g * _C, 8)

        def fire_gather(g):
            b = g % 2
            return [
                pltpu.async_copy(
                    table_hbm.at[idx[b].at[pl.ds(j * 128, 128)]],
                    rows[b].at[pl.ds(j * 128, 128)],
                    sg[b],
                )
                for j in range(_NSUB)
            ]

        def fire_wb(g):
            b = g % 2
            return pltpu.async_copy(rows[b], out_hbm.at[pl.ds(off(g), _C)], sw[b])

        pltpu.sync_copy(ids_hbm.at[pl.ds(off(0), _C)], idx0)
        gath = {0: fire_gather(0)}
        pltpu.sync_copy(ids_hbm.at[pl.ds(off(1), _C)], idx1)
        wb = {}
        for g in range(_NCHUNK):
            if g + 1 < _NCHUNK:
                if g - 1 in wb:
                    wb.pop(g - 1).wait()
                gath[g + 1] = fire_gather(g + 1)
            for cp in gath.pop(g):
                cp.wait()
            if g + 2 < _NCHUNK:
                pltpu.sync_copy(ids_hbm.at[pl.ds(off(g + 2), _C)], idx[g % 2])
            wb[g] = fire_wb(g)
        for g in sorted(wb):
            wb.pop(g).wait()

    return gather_kernel(ids_flat, table)


def _ln_body(w_ref, tid_ref, pos_ref, type_ref, gamma_ref, beta_ref, out_ref):
    w = w_ref[...]
    tid = tid_ref[:, 0, :]
    pos = pos_ref[...]
    t0 = type_ref[0:1, :]
    t1 = type_ref[1:2, :]
    t = jnp.where(tid[:, :, None] == 0, t0[None], t1[None])
    e = w + pos[None] + t
    mean = jnp.mean(e, axis=-1, keepdims=True)
    c = e - mean
    var = jnp.mean(c * c, axis=-1, keepdims=True)
    o = c * lax.rsqrt(var + 1e-12)
    out_ref[...] = o * gamma_ref[...][None] + beta_ref[...][None]


def _tc_ln(w, tid3, pos, typ, gamma, beta):
    return pl.pallas_call(
        _ln_body,
        out_shape=jax.ShapeDtypeStruct((_B, _L, _D), jnp.float32),
        grid=(_B // _BB,),
        in_specs=[
            pl.BlockSpec((_BB, _L, _D), lambda i: (i, 0, 0)),
            pl.BlockSpec((_BB, 1, _L), lambda i: (i, 0, 0)),
            pl.BlockSpec((_L, _D), lambda i: (0, 0)),
            pl.BlockSpec((8, _D), lambda i: (0, 0)),
            pl.BlockSpec((1, _D), lambda i: (0, 0)),
            pl.BlockSpec((1, _D), lambda i: (0, 0)),
        ],
        out_specs=pl.BlockSpec((_BB, _L, _D), lambda i: (i, 0, 0)),
    )(w, tid3, pos, typ, gamma, beta)


def kernel(input_ids, token_type_ids, word_emb, pos_emb, type_emb, ln_gamma, ln_beta):
    ids_flat = input_ids.reshape(_NTOK).astype(jnp.int32)
    w = _sc_gather(ids_flat, word_emb).reshape(_B, _L, _D)
    tid3 = token_type_ids.reshape(_B, 1, _L).astype(jnp.int32)
    typ8 = jnp.zeros((8, _D), jnp.float32).at[:2].set(type_emb)
    return _tc_ln(
        w,
        tid3,
        pos_emb[:_L],
        typ8,
        ln_gamma.reshape(1, _D),
        ln_beta.reshape(1, _D),
    )


# speedup vs baseline: 1.3075x; 1.0354x over previous
import functools

import jax
import jax.numpy as jnp
from jax import lax
from jax.experimental import pallas as pl
from jax.experimental.pallas import tpu as pltpu
from jax.experimental.pallas import tpu_sc as plsc

_B, _L, _D = 1024, 200, 128
_NTOK = _B * _L
_NW = 32
_TOK_PER_W = _NTOK // _NW
_C = 256
_NSUB = _C // 128
_NCHUNK = _TOK_PER_W // _C
_BB = 64


def _sc_gather(ids_flat, table):
    mesh = plsc.VectorSubcoreMesh(core_axis_name="c", subcore_axis_name="s")

    @functools.partial(
        pl.kernel,
        out_type=jax.ShapeDtypeStruct((_NTOK, _D), jnp.float32),
        mesh=mesh,
        scratch_types=[
            pltpu.VMEM((_C,), jnp.int32),
            pltpu.VMEM((_C,), jnp.int32),
            pltpu.VMEM((_C, _D), jnp.float32),
            pltpu.VMEM((_C, _D), jnp.float32),
            pltpu.SemaphoreType.DMA,
            pltpu.SemaphoreType.DMA,
            pltpu.SemaphoreType.DMA,
            pltpu.SemaphoreType.DMA,
        ],
    )
    def gather_kernel(ids_hbm, table_hbm, out_hbm, idx0, idx1, rows0, rows1,
                      sg0, sg1, sw0, sw1):
        wid = lax.axis_index("s") * 2 + lax.axis_index("c")
        base = wid * _TOK_PER_W
        idx = (idx0, idx1)
        rows = (rows0, rows1)
        sg = (sg0, sg1)
        sw = (sw0, sw1)

        def off(g):
            return pl.multiple_of(base + g * _C, 8)

        def fire_gather(g):
            b = g % 2
            return [
                pltpu.async_copy(
                    table_hbm.at[idx[b].at[pl.ds(j * 128, 128)]],
                    rows[b].at[pl.ds(j * 128, 128)],
                    sg[b],
                )
                for j in range(_NSUB)
            ]

        def fire_wb(g):
            b = g % 2
            return pltpu.async_copy(rows[b], out_hbm.at[pl.ds(off(g), _C)], sw[b])

        pltpu.sync_copy(ids_hbm.at[pl.ds(off(0), _C)], idx0)
        gath = {0: fire_gather(0)}
        pltpu.sync_copy(ids_hbm.at[pl.ds(off(1), _C)], idx1)
        wb = {}
        for g in range(_NCHUNK):
            if g + 1 < _NCHUNK:
                if g - 1 in wb:
                    wb.pop(g - 1).wait()
                gath[g + 1] = fire_gather(g + 1)
            for cp in gath.pop(g):
                cp.wait()
            if g + 2 < _NCHUNK:
                pltpu.sync_copy(ids_hbm.at[pl.ds(off(g + 2), _C)], idx[g % 2])
            wb[g] = fire_wb(g)
        for g in sorted(wb):
            wb.pop(g).wait()

    return gather_kernel(ids_flat, table)


def _ln_body(w_ref, tid_ref, pos_ref, type_ref, gamma_ref, beta_ref, out_ref):
    w = w_ref[...]
    tid = tid_ref[:, 0, :]
    pos = pos_ref[...]
    t0 = type_ref[0:1, :]
    t1 = type_ref[1:2, :]
    t = jnp.where(tid[:, :, None] == 0, t0[None], t1[None])
    e = w + pos[None] + t
    mean = jnp.mean(e, axis=-1, keepdims=True)
    c = e - mean
    var = jnp.mean(c * c, axis=-1, keepdims=True)
    o = c * lax.rsqrt(var + 1e-12)
    out_ref[...] = o * gamma_ref[...][None] + beta_ref[...][None]


def _tc_ln(w, tid3, pos, typ, gamma, beta):
    return pl.pallas_call(
        _ln_body,
        out_shape=jax.ShapeDtypeStruct((_B, _L, _D), jnp.float32),
        grid=(_B // _BB,),
        in_specs=[
            pl.BlockSpec((_BB, _L, _D), lambda i: (i, 0, 0)),
            pl.BlockSpec((_BB, 1, _L), lambda i: (i, 0, 0)),
            pl.BlockSpec((_L, _D), lambda i: (0, 0)),
            pl.BlockSpec((8, _D), lambda i: (0, 0)),
            pl.BlockSpec((1, _D), lambda i: (0, 0)),
            pl.BlockSpec((1, _D), lambda i: (0, 0)),
        ],
        out_specs=pl.BlockSpec((_BB, _L, _D), lambda i: (i, 0, 0)),
    )(w, tid3, pos, typ, gamma, beta)


def kernel(input_ids, token_type_ids, word_emb, pos_emb, type_emb, ln_gamma, ln_beta):
    ids_flat = input_ids.reshape(_NTOK).astype(jnp.int32)
    w = _sc_gather(ids_flat, word_emb).reshape(_B, _L, _D)
    tid3 = token_type_ids.reshape(_B, 1, _L).astype(jnp.int32)
    typ8 = jnp.zeros((8, _D), jnp.float32).at[:2].set(type_emb)
    return _tc_ln(
        w,
        tid3,
        pos_emb[:_L],
        typ8,
        ln_gamma.reshape(1, _D),
        ln_beta.reshape(1, _D),
    )


# speedup vs baseline: 1.3218x; 1.0109x over previous
import functools

import jax
import jax.numpy as jnp
from jax import lax
from jax.experimental import pallas as pl
from jax.experimental.pallas import tpu as pltpu
from jax.experimental.pallas import tpu_sc as plsc

_B, _L, _D = 1024, 200, 128
_NTOK = _B * _L
_NW = 32
_TOK_PER_W = _NTOK // _NW
_C = 256
_NSUB = _C // 128
_NCHUNK = _TOK_PER_W // _C
_BB = 64


def _sc_gather(ids_flat, table):
    mesh = plsc.VectorSubcoreMesh(core_axis_name="c", subcore_axis_name="s")

    @functools.partial(
        pl.kernel,
        out_type=jax.ShapeDtypeStruct((_NTOK, _D), jnp.float32),
        mesh=mesh,
        scratch_types=[
            pltpu.VMEM((_C,), jnp.int32),
            pltpu.VMEM((_C,), jnp.int32),
            pltpu.VMEM((_C,), jnp.int32),
            pltpu.VMEM((_C, _D), jnp.float32),
            pltpu.VMEM((_C, _D), jnp.float32),
            pltpu.VMEM((_C, _D), jnp.float32),
            pltpu.SemaphoreType.DMA,
            pltpu.SemaphoreType.DMA,
            pltpu.SemaphoreType.DMA,
            pltpu.SemaphoreType.DMA,
            pltpu.SemaphoreType.DMA,
            pltpu.SemaphoreType.DMA,
            pltpu.SemaphoreType.DMA,
            pltpu.SemaphoreType.DMA,
            pltpu.SemaphoreType.DMA,
        ],
    )
    def gather_kernel(ids_hbm, table_hbm, out_hbm, idx0, idx1, idx2,
                      rows0, rows1, rows2,
                      sg0, sg1, sg2, sw0, sw1, sw2, si0, si1, si2):
        wid = lax.axis_index("s") * 2 + lax.axis_index("c")
        base = wid * _TOK_PER_W
        idx = (idx0, idx1, idx2)
        rows = (rows0, rows1, rows2)
        sg = (sg0, sg1, sg2)
        sw = (sw0, sw1, sw2)
        si = (si0, si1, si2)

        def off(g):
            return pl.multiple_of(base + g * _C, 8)

        def fire_idx(g):
            b = g % 3
            return pltpu.async_copy(
                ids_hbm.at[pl.ds(off(g), _C)], idx[b], si[b]
            )

        def fire_gather(g):
            b = g % 3
            return [
                pltpu.async_copy(
                    table_hbm.at[idx[b].at[pl.ds(j * 128, 128)]],
                    rows[b].at[pl.ds(j * 128, 128)],
                    sg[b],
                )
                for j in range(_NSUB)
            ]

        def fire_wb(g):
            b = g % 3
            return pltpu.async_copy(rows[b], out_hbm.at[pl.ds(off(g), _C)], sw[b])

        icp = {0: fire_idx(0)}
        icp[0].wait()
        gath = {0: fire_gather(0)}
        if _NCHUNK > 1:
            icp[1] = fire_idx(1)
        wb = {}
        for g in range(_NCHUNK):
            if g + 1 < _NCHUNK:
                if g - 2 in wb:
                    wb.pop(g - 2).wait()
                icp.pop(g + 1).wait()
                gath[g + 1] = fire_gather(g + 1)
            if g + 2 < _NCHUNK:
                icp[g + 2] = fire_idx(g + 2)
            for cp in gath.pop(g):
                cp.wait()
            wb[g] = fire_wb(g)
        for g in sorted(wb):
            wb.pop(g).wait()

    return gather_kernel(ids_flat, table)


def _ln_body(w_ref, tid_ref, pos_ref, type_ref, gamma_ref, beta_ref, out_ref):
    w = w_ref[...]
    tid = tid_ref[:, 0, :]
    pos = pos_ref[...]
    t0 = type_ref[0:1, :]
    t1 = type_ref[1:2, :]
    t = jnp.where(tid[:, :, None] == 0, t0[None], t1[None])
    e = w + pos[None] + t
    mean = jnp.mean(e, axis=-1, keepdims=True)
    c = e - mean
    var = jnp.mean(c * c, axis=-1, keepdims=True)
    o = c * lax.rsqrt(var + 1e-12)
    out_ref[...] = o * gamma_ref[...][None] + beta_ref[...][None]


def _tc_ln(w, tid3, pos, typ, gamma, beta):
    return pl.pallas_call(
        _ln_body,
        out_shape=jax.ShapeDtypeStruct((_B, _L, _D), jnp.float32),
        grid=(_B // _BB,),
        in_specs=[
            pl.BlockSpec((_BB, _L, _D), lambda i: (i, 0, 0)),
            pl.BlockSpec((_BB, 1, _L), lambda i: (i, 0, 0)),
            pl.BlockSpec((_L, _D), lambda i: (0, 0)),
            pl.BlockSpec((8, _D), lambda i: (0, 0)),
            pl.BlockSpec((1, _D), lambda i: (0, 0)),
            pl.BlockSpec((1, _D), lambda i: (0, 0)),
        ],
        out_specs=pl.BlockSpec((_BB, _L, _D), lambda i: (i, 0, 0)),
    )(w, tid3, pos, typ, gamma, beta)


def kernel(input_ids, token_type_ids, word_emb, pos_emb, type_emb, ln_gamma, ln_beta):
    ids_flat = input_ids.reshape(_NTOK).astype(jnp.int32)
    w = _sc_gather(ids_flat, word_emb).reshape(_B, _L, _D)
    tid3 = token_type_ids.reshape(_B, 1, _L).astype(jnp.int32)
    typ8 = jnp.zeros((8, _D), jnp.float32).at[:2].set(type_emb)
    return _tc_ln(
        w,
        tid3,
        pos_emb[:_L],
        typ8,
        ln_gamma.reshape(1, _D),
        ln_beta.reshape(1, _D),
    )


# speedup vs baseline: 1.3768x; 1.0416x over previous
import functools

import jax
import jax.numpy as jnp
from jax import lax
from jax.experimental import pallas as pl
from jax.experimental.pallas import tpu as pltpu
from jax.experimental.pallas import tpu_sc as plsc

_B, _L, _D = 1024, 200, 128
_NTOK = _B * _L
_NW = 32
_C = 128
_NSUB = _C // 128
_NS = 2
_BS = _B // _NS
_TOKS = _NTOK // _NS
_BB = 64


def _sc_gather(ids_flat, table, ntok):
    mesh = plsc.VectorSubcoreMesh(core_axis_name="c", subcore_axis_name="s")
    tok_per_w = ntok // _NW
    nchunk = tok_per_w // _C

    @functools.partial(
        pl.kernel,
        out_type=jax.ShapeDtypeStruct((ntok, _D), jnp.float32),
        mesh=mesh,
        scratch_types=[
            pltpu.VMEM((_C,), jnp.int32),
            pltpu.VMEM((_C,), jnp.int32),
            pltpu.VMEM((_C,), jnp.int32),
            pltpu.VMEM((_C, _D), jnp.float32),
            pltpu.VMEM((_C, _D), jnp.float32),
            pltpu.VMEM((_C, _D), jnp.float32),
            pltpu.SemaphoreType.DMA,
            pltpu.SemaphoreType.DMA,
            pltpu.SemaphoreType.DMA,
            pltpu.SemaphoreType.DMA,
            pltpu.SemaphoreType.DMA,
            pltpu.SemaphoreType.DMA,
            pltpu.SemaphoreType.DMA,
            pltpu.SemaphoreType.DMA,
            pltpu.SemaphoreType.DMA,
        ],
    )
    def gather_kernel(ids_hbm, table_hbm, out_hbm, idx0, idx1, idx2,
                      rows0, rows1, rows2,
                      sg0, sg1, sg2, sw0, sw1, sw2, si0, si1, si2):
        wid = lax.axis_index("s") * 2 + lax.axis_index("c")
        base = wid * tok_per_w
        idx = (idx0, idx1, idx2)
        rows = (rows0, rows1, rows2)
        sg = (sg0, sg1, sg2)
        sw = (sw0, sw1, sw2)
        si = (si0, si1, si2)

        def off(g):
            return pl.multiple_of(base + g * _C, 8)

        def fire_idx(g):
            b = g % 3
            return pltpu.async_copy(
                ids_hbm.at[pl.ds(off(g), _C)], idx[b], si[b]
            )

        def fire_gather(g):
            b = g % 3
            return [
                pltpu.async_copy(
                    table_hbm.at[idx[b].at[pl.ds(j * 128, 128)]],
                    rows[b].at[pl.ds(j * 128, 128)],
                    sg[b],
                )
                for j in range(_NSUB)
            ]

        def fire_wb(g):
            b = g % 3
            return pltpu.async_copy(rows[b], out_hbm.at[pl.ds(off(g), _C)], sw[b])

        icp = {0: fire_idx(0)}
        icp[0].wait()
        gath = {0: fire_gather(0)}
        if nchunk > 1:
            icp[1] = fire_idx(1)
        wb = {}
        for g in range(nchunk):
            if g + 1 < nchunk:
                if g - 2 in wb:
                    wb.pop(g - 2).wait()
                icp.pop(g + 1).wait()
                gath[g + 1] = fire_gather(g + 1)
            if g + 2 < nchunk:
                icp[g + 2] = fire_idx(g + 2)
            for cp in gath.pop(g):
                cp.wait()
            wb[g] = fire_wb(g)
        for g in sorted(wb):
            wb.pop(g).wait()

    return gather_kernel(ids_flat, table)


def _ln_body(w_ref, tid_ref, pos_ref, type_ref, gamma_ref, beta_ref, out_ref):
    w = w_ref[...]
    tid = tid_ref[:, 0, :]
    pos = pos_ref[...]
    t0 = type_ref[0:1, :]
    t1 = type_ref[1:2, :]
    t = jnp.where(tid[:, :, None] == 0, t0[None], t1[None])
    e = w + pos[None] + t
    mean = jnp.mean(e, axis=-1, keepdims=True)
    c = e - mean
    var = jnp.mean(c * c, axis=-1, keepdims=True)
    o = c * lax.rsqrt(var + 1e-12)
    out_ref[...] = o * gamma_ref[...][None] + beta_ref[...][None]


def _ln_body_alias(w_ref, tid_ref, pos_ref, type_ref, gamma_ref, beta_ref,
                   prev_ref, out_ref):
    del prev_ref
    _ln_body(w_ref, tid_ref, pos_ref, type_ref, gamma_ref, beta_ref, out_ref)


def _tc_ln_slice(w_s, tid3_s, pos, typ, gamma, beta, out_prev, s):
    nb = _BS // _BB
    specs = [
        pl.BlockSpec((_BB, _L, _D), lambda i: (i, 0, 0)),
        pl.BlockSpec((_BB, 1, _L), lambda i: (i, 0, 0)),
        pl.BlockSpec((_L, _D), lambda i: (0, 0)),
        pl.BlockSpec((8, _D), lambda i: (0, 0)),
        pl.BlockSpec((1, _D), lambda i: (0, 0)),
        pl.BlockSpec((1, _D), lambda i: (0, 0)),
    ]
    args = [w_s, tid3_s, pos, typ, gamma, beta]
    kwargs = {}
    body = _ln_body
    if out_prev is not None:
        specs.append(pl.BlockSpec(memory_space=pl.ANY))
        args.append(out_prev)
        kwargs = dict(input_output_aliases={6: 0})
        body = _ln_body_alias
    return pl.pallas_call(
        body,
        out_shape=jax.ShapeDtypeStruct((_B, _L, _D), jnp.float32),
        grid=(nb,),
        in_specs=specs,
        out_specs=pl.BlockSpec((_BB, _L, _D), lambda i, s=s: (i + s * nb, 0, 0)),
        **kwargs,
    )(*args)


def kernel(input_ids, token_type_ids, word_emb, pos_emb, type_emb, ln_gamma, ln_beta):
    ids_flat = input_ids.reshape(_NTOK).astype(jnp.int32)
    tid3 = token_type_ids.reshape(_B, 1, _L).astype(jnp.int32)
    typ8 = jnp.zeros((8, _D), jnp.float32).at[:2].set(type_emb)
    pos = pos_emb[:_L]
    gamma = ln_gamma.reshape(1, _D)
    beta = ln_beta.reshape(1, _D)

    ws = [
        _sc_gather(ids_flat[s * _TOKS:(s + 1) * _TOKS], word_emb, _TOKS)
        .reshape(_BS, _L, _D)
        for s in range(_NS)
    ]
    out = None
    for s in range(_NS):
        out = _tc_ln_slice(
            ws[s], tid3[s * _BS:(s + 1) * _BS], pos, typ8, gamma, beta, out, s
        )
    return out
